# Initial kernel scaffold; baseline (speedup 1.0000x reference)
import jax
import jax.numpy as jnp
from jax.experimental import pallas as pl


def kernel(user_id, ad_id, user_table, ad_table, W, b):
    raise NotImplementedError("write your pallas kernel here")


# speedup vs baseline: 1.9313x; 1.9313x over previous
import functools

import jax
import jax.numpy as jnp
from jax import lax
from jax.experimental import pallas as pl
from jax.experimental.pallas import tpu as pltpu
from jax.experimental.pallas import tpu_sc as plsc

NC = 2
NS = 16
NW = NC * NS
LANES = 16

CB = 16


def _sc_dot(uid_flat, aid_flat, user_table, ad_table, B, L, E):
    IDX = CB * L
    rows_per_w = B // NW
    chunks = rows_per_w // CB
    assert rows_per_w % CB == 0
    n_lane_grp = E // LANES

    mesh = plsc.VectorSubcoreMesh(core_axis_name="c", subcore_axis_name="s")

    @functools.partial(
        pl.kernel,
        mesh=mesh,
        compiler_params=pltpu.CompilerParams(use_tc_tiling_on_sc=False),
        out_type=jax.ShapeDtypeStruct((B, E), jnp.float32),
        scratch_types=[
            pltpu.VMEM((IDX,), jnp.int32),
            pltpu.VMEM((IDX,), jnp.int32),
            pltpu.VMEM((IDX, E), jnp.float32),
            pltpu.VMEM((IDX, E), jnp.float32),
            pltpu.VMEM((CB, E), jnp.float32),
            pltpu.SemaphoreType.DMA,
        ],
    )
    def sc_kernel(uid_hbm, aid_hbm, utab_hbm, atab_hbm, out_hbm,
                  uidx_v, aidx_v, u_v, a_v, o_v, sem):
        wid = lax.axis_index("s") * NC + lax.axis_index("c")
        base_row = wid * rows_per_w

        @pl.loop(0, chunks)
        def _(chunk):
            row0 = base_row + chunk * CB
            i0 = row0 * L
            pltpu.sync_copy(uid_hbm.at[pl.ds(i0, IDX)], uidx_v)
            pltpu.sync_copy(aid_hbm.at[pl.ds(i0, IDX)], aidx_v)
            copies = []
            for k in range(0, IDX, 128):
                n = min(128, IDX - k)
                copies.append(pltpu.async_copy(
                    utab_hbm.at[uidx_v.at[pl.ds(k, n)]],
                    u_v.at[pl.ds(k, n)], sem))
                copies.append(pltpu.async_copy(
                    atab_hbm.at[aidx_v.at[pl.ds(k, n)]],
                    a_v.at[pl.ds(k, n)], sem))
            for cp in copies:
                cp.wait()

            @pl.loop(0, CB)
            def _(i):
                r0 = i * L
                for c in range(n_lane_grp):
                    sl = pl.ds(c * LANES, LANES)

                    def body(l, acc):
                        return acc + u_v[r0 + l, sl] * a_v[r0 + l, sl]

                    acc = lax.fori_loop(
                        0, L, body, jnp.zeros((LANES,), jnp.float32))
                    o_v[i, sl] = acc

            pltpu.sync_copy(o_v, out_hbm.at[pl.ds(row0, CB)])

    return sc_kernel(uid_flat, aid_flat, user_table, ad_table)


def _tc_head(dot, W, b, B, E):
    BLK = 1024

    def body(d_ref, w_ref, b_ref, o_ref):
        s = jnp.dot(d_ref[...], w_ref[...],
                    preferred_element_type=jnp.float32)
        o_ref[...] = jax.nn.sigmoid(s + b_ref[0, 0])

    return pl.pallas_call(
        body,
        grid=(B // BLK,),
        in_specs=[
            pl.BlockSpec((BLK, E), lambda i: (i, 0)),
            pl.BlockSpec((E, 1), lambda i: (0, 0)),
            pl.BlockSpec((1, 1), lambda i: (0, 0)),
        ],
        out_specs=pl.BlockSpec((BLK, 1), lambda i: (i, 0)),
        out_shape=jax.ShapeDtypeStruct((B, 1), jnp.float32),
    )(dot, W, b.reshape(1, 1))


def kernel(user_id, ad_id, user_table, ad_table, W, b):
    B, L = user_id.shape
    E = user_table.shape[1]
    dot = _sc_dot(user_id.reshape(-1), ad_id.reshape(-1),
                  user_table, ad_table, B, L, E)
    return _tc_head(dot, W, b, B, E)


# speedup vs baseline: 2.4312x; 1.2588x over previous
import functools

import jax
import jax.numpy as jnp
from jax import lax
from jax.experimental import pallas as pl
from jax.experimental.pallas import tpu as pltpu
from jax.experimental.pallas import tpu_sc as plsc

NC = 2
NS = 16
NW = NC * NS
LANES = 16

CB = 16


def _sc_dot(uid_flat, aid_flat, user_table, ad_table, B, L, E):
    IDX = CB * L
    rows_per_w = B // NW
    chunks = rows_per_w // CB
    idx_per_w = rows_per_w * L
    assert rows_per_w % CB == 0 and chunks % 2 == 0
    n_lane_grp = E // LANES

    mesh = plsc.VectorSubcoreMesh(core_axis_name="c", subcore_axis_name="s")

    @functools.partial(
        pl.kernel,
        mesh=mesh,
        compiler_params=pltpu.CompilerParams(use_tc_tiling_on_sc=False),
        out_type=jax.ShapeDtypeStruct((B, E), jnp.float32),
        scratch_types=[
            pltpu.VMEM((idx_per_w,), jnp.int32),
            pltpu.VMEM((idx_per_w,), jnp.int32),
            pltpu.VMEM((IDX, E), jnp.float32),
            pltpu.VMEM((IDX, E), jnp.float32),
            pltpu.VMEM((IDX, E), jnp.float32),
            pltpu.VMEM((IDX, E), jnp.float32),
            pltpu.VMEM((CB, E), jnp.float32),
            pltpu.VMEM((CB, E), jnp.float32),
            pltpu.SemaphoreType.DMA,
            pltpu.SemaphoreType.DMA,
        ],
    )
    def sc_kernel(uid_hbm, aid_hbm, utab_hbm, atab_hbm, out_hbm,
                  uidx_v, aidx_v, u0, a0, u1, a1, o0, o1, sem0, sem1):
        wid = lax.axis_index("s") * NC + lax.axis_index("c")
        base_row = wid * rows_per_w
        base_idx = base_row * L

        pltpu.sync_copy(uid_hbm.at[pl.ds(base_idx, idx_per_w)], uidx_v)
        pltpu.sync_copy(aid_hbm.at[pl.ds(base_idx, idx_per_w)], aidx_v)

        def start(chunk, u_v, a_v, sem):
            c0 = chunk * IDX
            for k in range(0, IDX, 128):
                n = min(128, IDX - k)
                pltpu.async_copy(
                    utab_hbm.at[uidx_v.at[pl.ds(c0 + k, n)]],
                    u_v.at[pl.ds(k, n)], sem)
                pltpu.async_copy(
                    atab_hbm.at[aidx_v.at[pl.ds(c0 + k, n)]],
                    a_v.at[pl.ds(k, n)], sem)

        def drain(chunk, u_v, a_v, sem):
            c0 = chunk * IDX
            for k in range(0, IDX, 128):
                n = min(128, IDX - k)
                pltpu.make_async_copy(
                    utab_hbm.at[uidx_v.at[pl.ds(c0 + k, n)]],
                    u_v.at[pl.ds(k, n)], sem).wait()
                pltpu.make_async_copy(
                    atab_hbm.at[aidx_v.at[pl.ds(c0 + k, n)]],
                    a_v.at[pl.ds(k, n)], sem).wait()

        def compute(u_v, a_v, o_v):
            @pl.loop(0, CB)
            def _(i):
                r0 = i * L
                for c in range(n_lane_grp):
                    sl = pl.ds(c * LANES, LANES)
                    acc = u_v[r0, sl] * a_v[r0, sl]
                    for l in range(1, L):
                        acc = acc + u_v[r0 + l, sl] * a_v[r0 + l, sl]
                    o_v[i, sl] = acc

        start(0, u0, a0, sem0)

        @pl.loop(0, chunks, step=2)
        def _(g):
            row0 = base_row + g * CB
            start(g + 1, u1, a1, sem1)
            drain(g, u0, a0, sem0)
            compute(u0, a0, o0)
            pltpu.sync_copy(o0, out_hbm.at[pl.ds(row0, CB)])

            @pl.when(g + 2 < chunks)
            def _():
                start(g + 2, u0, a0, sem0)

            drain(g + 1, u1, a1, sem1)
            compute(u1, a1, o1)
            pltpu.sync_copy(o1, out_hbm.at[pl.ds(row0 + CB, CB)])

    return sc_kernel(uid_flat, aid_flat, user_table, ad_table)


def _tc_head(dot, W, b, B, E):
    BLK = 1024

    def body(d_ref, w_ref, b_ref, o_ref):
        s = jnp.dot(d_ref[...], w_ref[...],
                    preferred_element_type=jnp.float32)
        o_ref[...] = jax.nn.sigmoid(s + b_ref[0, 0])

    return pl.pallas_call(
        body,
        grid=(B // BLK,),
        in_specs=[
            pl.BlockSpec((BLK, E), lambda i: (i, 0)),
            pl.BlockSpec((E, 1), lambda i: (0, 0)),
            pl.BlockSpec((1, 1), lambda i: (0, 0)),
        ],
        out_specs=pl.BlockSpec((BLK, 1), lambda i: (i, 0)),
        out_shape=jax.ShapeDtypeStruct((B, 1), jnp.float32),
    )(dot, W, b.reshape(1, 1))


def kernel(user_id, ad_id, user_table, ad_table, W, b):
    B, L = user_id.shape
    E = user_table.shape[1]
    dot = _sc_dot(user_id.reshape(-1), ad_id.reshape(-1),
                  user_table, ad_table, B, L, E)
    return _tc_head(dot, W, b, B, E)
